# baseline (speedup 1.0000x reference)
import numpy as np
import jax
import jax.numpy as jnp
from jax.experimental import pallas as pl
from jax.experimental.pallas import tpu as pltpu

HIDDEN = 512
HEADS = 8
DHEAD = 64
TOPK = 8
CHUNK = 32
NQ = 128
NCHUNK = 625
G = 4
STEPS = NQ // G
ROWS = G * HEADS
SEL = G * TOPK
TOK = SEL * CHUNK
U = 8
NSTEP = STEPS // U
NEG = -3.4028235e38
JB = 8
PB = CHUNK // JB
HIGHEST = jax.lax.Precision.HIGHEST
HIGH = jax.lax.Precision.HIGH


def _bfdot(a, b):
    return jax.lax.dot_general(
        a.astype(jnp.bfloat16), b.astype(jnp.bfloat16),
        (((1,), (0,)), ((), ())), preferred_element_type=jnp.float32)


def _bfdot_t(a, b):
    return jax.lax.dot_general(
        a.astype(jnp.bfloat16), b.astype(jnp.bfloat16),
        (((1,), (1,)), ((), ())), preferred_element_type=jnp.float32)


def _bfdot16(a, b):
    return jax.lax.dot_general(
        a.astype(jnp.bfloat16), b.astype(jnp.bfloat16),
        (((1,), (0,)), ((), ())), preferred_element_type=jnp.bfloat16)


def _bfdot16_t(a, b):
    return jax.lax.dot_general(
        a.astype(jnp.bfloat16), b.astype(jnp.bfloat16),
        (((1,), (1,)), ((), ())), preferred_element_type=jnp.bfloat16)


def _score_kernel(mem3, q2, sqw, sqb, skw, skb, tqw, tkv, pos,
                  qk_out, qkpos_out, idx_out, w_out, accr):
    b = pl.program_id(0)

    @pl.when(b == 0)
    def _init():
        accr[...] = jnp.zeros((NCHUNK, HIDDEN), jnp.float32)

    @pl.when(b < PB)
    def _pool():
        blk = mem3[...]
        part = blk[:, 0, :]
        for j in range(1, JB):
            part = part + blk[:, j, :]
        accr[...] = accr[...] + part

    @pl.when(b == PB)
    def _rest():
        _score_tail(q2, sqw, sqb, skw, skb, tqw, tkv, pos,
                    qk_out, qkpos_out, idx_out, w_out, accr)


def _score_tail(q2, sqw, sqb, skw, skb, tqw, tkv, pos,
                qk_out, qkpos_out, idx_out, w_out, accr):
    summar = accr[...] / (np.float32(CHUNK) + np.float32(1e-5))

    q2v = q2[...]
    sq = _bfdot(q2v, sqw[...]) + sqb[...]
    sk = _bfdot(summar, skw[...]) + skb[...]
    sim = _bfdot_t(sq, sk) * np.float32(HIDDEN ** -0.5)

    iota = jax.lax.broadcasted_iota(jnp.int32, (NQ, NCHUNK), 1)
    s = sim
    logits = []
    idxs = []
    for _ in range(TOPK):
        m = jnp.max(s, axis=1, keepdims=True)
        cand = jnp.where(s >= m, iota, jnp.int32(2 ** 30))
        ix = jnp.min(cand, axis=1, keepdims=True)
        s = jnp.where(iota == ix, jnp.float32(NEG), s)
        logits.append(m)
        idxs.append(ix)
    lg = jnp.concatenate(logits, axis=1)
    e = jnp.exp(lg - lg[:, 0:1])
    w_out[...] = e / jnp.sum(e, axis=1, keepdims=True)
    idx_out[...] = jnp.concatenate(idxs, axis=1)

    qall = _bfdot(q2v, tqw[...]) * np.float32(DHEAD ** -0.5)
    wkv = tkv[...]
    posv = pos[...]
    for h in range(HEADS):
        qk_h = _bfdot_t(qall[:, h * DHEAD:(h + 1) * DHEAD],
                        wkv[:, h * DHEAD:(h + 1) * DHEAD])
        qk_out[:, h, :] = qk_h.astype(jnp.bfloat16)
        qkpos_out[:, h, :] = _bfdot_t(qk_h, posv).astype(jnp.bfloat16)


def _attn_kernel(idx_ref, memhbm, qk_ref, qkpos_ref, w_ref, pos_ref,
                 seg_ref, segt_ref, tc_ref, tct_ref,
                 wv_ref, wo_ref, wob_ref, f2w_ref, f2b_ref,
                 out_ref, stk, o1acc, gsem):
    s = pl.program_id(0)

    def issue(step, slot):
        for u in range(U):
            for t in range(SEL):
                c = idx_ref[(step * U + u) * SEL + t]
                pltpu.make_async_copy(
                    memhbm.at[pl.ds(c * CHUNK, CHUNK), :],
                    stk.at[slot, u, pl.ds(t * CHUNK, CHUNK), :],
                    gsem.at[slot, u]).start()

    @pl.when(s == 0)
    def _first():
        issue(0, 0)

    @pl.when(s < NSTEP - 1)
    def _prefetch():
        issue(s + 1, (s + 1) % 2)

    slot = s % 2
    qkall = qk_ref[...]
    qkposall = qkpos_ref[...]
    wall = w_ref[...].reshape(1, U * SEL)
    segv = seg_ref[...]
    segtv = segt_ref[...]
    tcv = tc_ref[...]
    tctv = tct_ref[...]
    posv = pos_ref[...]
    wvv = wv_ref[...]

    ri = jax.lax.broadcasted_iota(jnp.int32, (ROWS, SEL), 0)
    ci = jax.lax.broadcasted_iota(jnp.int32, (ROWS, SEL), 1)
    same_g = (ri // HEADS) == (ci // TOPK)
    r2 = jax.lax.broadcasted_iota(jnp.int32, (ROWS, HIDDEN), 0)
    c2 = jax.lax.broadcasted_iota(jnp.int32, (ROWS, HIDDEN), 1)
    keep = (r2 % HEADS) == (c2 // DHEAD)

    for u in range(U):
        for t in range(SEL):
            pltpu.make_async_copy(
                memhbm.at[pl.ds(0, CHUNK), :],
                stk.at[0, 0, pl.ds(t * CHUNK, CHUNK), :],
                gsem.at[slot, u]).wait()

    for u in range(U):
        st = stk[slot, u]
        qk = qkall[u * ROWS:(u + 1) * ROWS, :]
        qkpos_t = _bfdot(qkposall[u * ROWS:(u + 1) * ROWS, :], tctv)
        scores = _bfdot_t(qk, st) + qkpos_t
        rowmax = jnp.max(scores, axis=1, keepdims=True)
        ex = jnp.exp(scores - rowmax)
        denom = _bfdot(ex, segv)
        wrow = jnp.broadcast_to(wall[:, u * SEL:(u + 1) * SEL], (ROWS, SEL))
        fac = jnp.where(same_g, wrow / jnp.maximum(denom, 1e-20), 0.0)
        spread = _bfdot(fac, segtv)
        attnw = ex * spread
        colsum = _bfdot(attnw, tcv)
        pool = _bfdot(attnw, st) + _bfdot(colsum, posv)
        z = _bfdot(pool, wvv)
        o1 = jnp.where(keep, z, 0.0).reshape(G, HEADS, HIDDEN).sum(axis=1)
        o1acc[s, u, pl.ds(0, G), :] = o1

    @pl.when(s == NSTEP - 1)
    def _finish():
        o1full = o1acc[...][:, :, 0:G, :].reshape(NQ, HIDDEN)
        htm = _bfdot(o1full, wo_ref[...]) + wob_ref[...]
        out_ref[...] = _bfdot(htm, f2w_ref[...]) + f2b_ref[...]


def _pos_emb():
    freqs = np.arange(0, HIDDEN, 2.0)
    inv = 10000.0 ** (-freqs / HIDDEN)
    seq = np.arange(CHUNK - 1, -1, -1.0)
    si = seq[:, None] * inv[None, :]
    return np.concatenate([np.sin(si), np.cos(si)], axis=-1).astype(np.float32)


@jax.jit
def _run(queries, memories, sq_w, sq_b, sk_w, sk_b, to_q_w, to_kv_w,
         to_out_w, to_out_b, fc2_w, fc2_b):
    q2 = queries[0]
    mem3 = memories[0].reshape(NCHUNK, CHUNK, HIDDEN)
    mem2 = memories[0]
    wv = to_kv_w[:, HIDDEN:]
    pos = jnp.asarray(_pos_emb())

    qk, qkpos, idx8, w8 = pl.pallas_call(
        _score_kernel,
        grid=(PB + 1,),
        in_specs=[
            pl.BlockSpec((NCHUNK, JB, HIDDEN),
                         lambda b: (0, jnp.minimum(b, PB - 1), 0)),
            pl.BlockSpec((NQ, HIDDEN), lambda b: (0, 0)),
            pl.BlockSpec((HIDDEN, HIDDEN), lambda b: (0, 0)),
            pl.BlockSpec((1, HIDDEN), lambda b: (0, 0)),
            pl.BlockSpec((HIDDEN, HIDDEN), lambda b: (0, 0)),
            pl.BlockSpec((1, HIDDEN), lambda b: (0, 0)),
            pl.BlockSpec((HIDDEN, HIDDEN), lambda b: (0, 0)),
            pl.BlockSpec((HIDDEN, HIDDEN), lambda b: (0, 0)),
            pl.BlockSpec((CHUNK, HIDDEN), lambda b: (0, 0)),
        ],
        out_specs=[
            pl.BlockSpec((NQ, HEADS, HIDDEN), lambda b: (0, 0, 0)),
            pl.BlockSpec((NQ, HEADS, CHUNK), lambda b: (0, 0, 0)),
            pl.BlockSpec((NQ, TOPK), lambda b: (0, 0)),
            pl.BlockSpec((NQ, TOPK), lambda b: (0, 0)),
        ],
        scratch_shapes=[pltpu.VMEM((NCHUNK, HIDDEN), jnp.float32)],
        out_shape=[
            jax.ShapeDtypeStruct((NQ, HEADS, HIDDEN), jnp.bfloat16),
            jax.ShapeDtypeStruct((NQ, HEADS, CHUNK), jnp.bfloat16),
            jax.ShapeDtypeStruct((NQ, TOPK), jnp.int32),
            jax.ShapeDtypeStruct((NQ, TOPK), jnp.float32),
        ],
    )(mem3, q2, sq_w, sq_b.reshape(1, HIDDEN), sk_w, sk_b.reshape(1, HIDDEN),
      to_q_w, to_kv_w[:, :HIDDEN], pos)

    qk2 = qk.reshape(NQ * HEADS, HIDDEN)
    qkpos2 = qkpos.reshape(NQ * HEADS, CHUNK)
    idx_flat = idx8.reshape(NQ * TOPK)
    w3 = w8.reshape(NSTEP, 1, U * SEL)

    seg = jnp.asarray(
        (np.arange(TOK)[:, None] // CHUNK == np.arange(SEL)[None, :]
         ).astype(np.float32)).astype(jnp.bfloat16)
    tc = jnp.asarray(
        (np.arange(TOK)[:, None] % CHUNK == np.arange(CHUNK)[None, :]
         ).astype(np.float32)).astype(jnp.bfloat16)
    segt = seg.T
    tct = tc.T
    pos16 = pos.astype(jnp.bfloat16)
    wv16 = wv.astype(jnp.bfloat16)
    wo16 = to_out_w.astype(jnp.bfloat16)
    f2w16 = fc2_w.astype(jnp.bfloat16)

    grid_spec = pltpu.PrefetchScalarGridSpec(
        num_scalar_prefetch=1,
        grid=(NSTEP,),
        in_specs=[
            pl.BlockSpec(memory_space=pl.ANY),
            pl.BlockSpec((U * ROWS, HIDDEN), lambda s, n: (s, 0)),
            pl.BlockSpec((U * ROWS, CHUNK), lambda s, n: (s, 0)),
            pl.BlockSpec((1, 1, U * SEL), lambda s, n: (s, 0, 0)),
            pl.BlockSpec((CHUNK, HIDDEN), lambda s, n: (0, 0)),
            pl.BlockSpec((TOK, SEL), lambda s, n: (0, 0)),
            pl.BlockSpec((SEL, TOK), lambda s, n: (0, 0)),
            pl.BlockSpec((TOK, CHUNK), lambda s, n: (0, 0)),
            pl.BlockSpec((CHUNK, TOK), lambda s, n: (0, 0)),
            pl.BlockSpec((HIDDEN, HIDDEN), lambda s, n: (0, 0)),
            pl.BlockSpec((HIDDEN, HIDDEN), lambda s, n: (0, 0)),
            pl.BlockSpec((1, HIDDEN), lambda s, n: (0, 0)),
            pl.BlockSpec((HIDDEN, 5), lambda s, n: (0, 0)),
            pl.BlockSpec((1, 5), lambda s, n: (0, 0)),
        ],
        out_specs=pl.BlockSpec((NQ, 5), lambda s, n: (0, 0)),
        scratch_shapes=[
            pltpu.VMEM((2, U, TOK, HIDDEN), jnp.float32),
            pltpu.VMEM((NSTEP, U, 8, HIDDEN), jnp.float32),
            pltpu.SemaphoreType.DMA((2, U)),
        ],
    )

    out = pl.pallas_call(
        _attn_kernel,
        grid_spec=grid_spec,
        out_shape=jax.ShapeDtypeStruct((NQ, 5), jnp.float32),
    )(idx_flat, mem2, qk2, qkpos2, w3, pos16, seg, segt, tc, tct, wv16, wo16,
      to_out_b.reshape(1, HIDDEN), f2w16, fc2_b.reshape(1, 5))

    return out.reshape(1, NQ, 5)


def kernel(x, queries, memories, conv1_w, conv1_b, conv2_w, conv2_b,
           fc1_w, fc1_b, sq_w, sq_b, sk_w, sk_b, to_q_w, to_kv_w,
           to_out_w, to_out_b, fc2_w, fc2_b, mask):
    return _run(queries, memories, sq_w, sq_b, sk_w, sk_b, to_q_w, to_kv_w,
                to_out_w, to_out_b, fc2_w, fc2_b)


# speedup vs baseline: 1.0081x; 1.0081x over previous
import numpy as np
import jax
import jax.numpy as jnp
from jax.experimental import pallas as pl
from jax.experimental.pallas import tpu as pltpu

HIDDEN = 512
HEADS = 8
DHEAD = 64
TOPK = 8
CHUNK = 32
NQ = 128
NCHUNK = 625
G = 4
STEPS = NQ // G
ROWS = G * HEADS
SEL = G * TOPK
TOK = SEL * CHUNK
U = 4
NSTEP = STEPS // U
NEG = -3.4028235e38
JB = 8
PB = CHUNK // JB
HIGHEST = jax.lax.Precision.HIGHEST
HIGH = jax.lax.Precision.HIGH


def _bfdot(a, b):
    return jax.lax.dot_general(
        a.astype(jnp.bfloat16), b.astype(jnp.bfloat16),
        (((1,), (0,)), ((), ())), preferred_element_type=jnp.float32)


def _bfdot_t(a, b):
    return jax.lax.dot_general(
        a.astype(jnp.bfloat16), b.astype(jnp.bfloat16),
        (((1,), (1,)), ((), ())), preferred_element_type=jnp.float32)


def _bfdot16(a, b):
    return jax.lax.dot_general(
        a.astype(jnp.bfloat16), b.astype(jnp.bfloat16),
        (((1,), (0,)), ((), ())), preferred_element_type=jnp.bfloat16)


def _bfdot16_t(a, b):
    return jax.lax.dot_general(
        a.astype(jnp.bfloat16), b.astype(jnp.bfloat16),
        (((1,), (1,)), ((), ())), preferred_element_type=jnp.bfloat16)


def _score_kernel(mem3, q2, sqw, sqb, skw, skb, tqw, tkv, pos,
                  qk_out, qkpos_out, idx_out, w_out, accr):
    b = pl.program_id(0)

    @pl.when(b == 0)
    def _init():
        accr[...] = jnp.zeros((NCHUNK, HIDDEN), jnp.float32)

    @pl.when(b < PB)
    def _pool():
        blk = mem3[...]
        part = blk[:, 0, :]
        for j in range(1, JB):
            part = part + blk[:, j, :]
        accr[...] = accr[...] + part

    @pl.when(b == PB)
    def _rest():
        _score_tail(q2, sqw, sqb, skw, skb, tqw, tkv, pos,
                    qk_out, qkpos_out, idx_out, w_out, accr)


def _score_tail(q2, sqw, sqb, skw, skb, tqw, tkv, pos,
                qk_out, qkpos_out, idx_out, w_out, accr):
    summar = accr[...] / (np.float32(CHUNK) + np.float32(1e-5))

    q2v = q2[...]
    sq = _bfdot(q2v, sqw[...]) + sqb[...]
    sk = _bfdot(summar, skw[...]) + skb[...]
    sim = _bfdot_t(sq, sk) * np.float32(HIDDEN ** -0.5)

    iota = jax.lax.broadcasted_iota(jnp.int32, (NQ, NCHUNK), 1)
    s = sim
    logits = []
    idxs = []
    for _ in range(TOPK):
        m = jnp.max(s, axis=1, keepdims=True)
        cand = jnp.where(s >= m, iota, jnp.int32(2 ** 30))
        ix = jnp.min(cand, axis=1, keepdims=True)
        s = jnp.where(iota == ix, jnp.float32(NEG), s)
        logits.append(m)
        idxs.append(ix)
    lg = jnp.concatenate(logits, axis=1)
    e = jnp.exp(lg - lg[:, 0:1])
    w_out[...] = e / jnp.sum(e, axis=1, keepdims=True)
    idx_out[...] = jnp.concatenate(idxs, axis=1)

    qall = _bfdot(q2v, tqw[...]) * np.float32(DHEAD ** -0.5)
    wkv = tkv[...]
    posv = pos[...]
    for h in range(HEADS):
        qk_h = _bfdot_t(qall[:, h * DHEAD:(h + 1) * DHEAD],
                        wkv[:, h * DHEAD:(h + 1) * DHEAD])
        qk_out[:, h, :] = qk_h.astype(jnp.bfloat16)
        qkpos_out[:, h, :] = _bfdot_t(qk_h, posv).astype(jnp.bfloat16)


def _attn_kernel(idx_ref, memhbm, qk_ref, qkpos_ref, w_ref, pos_ref,
                 seg_ref, segt_ref, tc_ref, tct_ref,
                 wv_ref, wo_ref, wob_ref, f2w_ref, f2b_ref,
                 out_ref, stk, o1acc, gsem):
    s = pl.program_id(0)

    def issue(step, slot):
        for u in range(U):
            for t in range(SEL):
                c = idx_ref[(step * U + u) * SEL + t]
                pltpu.make_async_copy(
                    memhbm.at[pl.ds(c * CHUNK, CHUNK), :],
                    stk.at[slot, u, pl.ds(t * CHUNK, CHUNK), :],
                    gsem.at[slot, u]).start()

    @pl.when(s == 0)
    def _first():
        issue(0, 0)

    @pl.when(s < NSTEP - 1)
    def _prefetch():
        issue(s + 1, (s + 1) % 2)

    slot = s % 2
    qkall = qk_ref[...]
    qkposall = qkpos_ref[...]
    wall = w_ref[...].reshape(1, U * SEL)
    segv = seg_ref[...]
    segtv = segt_ref[...]
    tcv = tc_ref[...]
    tctv = tct_ref[...]
    posv = pos_ref[...]
    wvv = wv_ref[...]

    ri = jax.lax.broadcasted_iota(jnp.int32, (ROWS, SEL), 0)
    ci = jax.lax.broadcasted_iota(jnp.int32, (ROWS, SEL), 1)
    same_g = (ri // HEADS) == (ci // TOPK)
    r2 = jax.lax.broadcasted_iota(jnp.int32, (ROWS, HIDDEN), 0)
    c2 = jax.lax.broadcasted_iota(jnp.int32, (ROWS, HIDDEN), 1)
    keep = (r2 % HEADS) == (c2 // DHEAD)

    for u in range(U):
        for t in range(SEL):
            pltpu.make_async_copy(
                memhbm.at[pl.ds(0, CHUNK), :],
                stk.at[0, 0, pl.ds(t * CHUNK, CHUNK), :],
                gsem.at[slot, u]).wait()

    for u in range(U):
        st = stk[slot, u]
        qk = qkall[u * ROWS:(u + 1) * ROWS, :]
        qkpos_t = _bfdot(qkposall[u * ROWS:(u + 1) * ROWS, :], tctv)
        scores = _bfdot_t(qk, st) + qkpos_t
        rowmax = jnp.max(scores, axis=1, keepdims=True)
        ex = jnp.exp(scores - rowmax)
        denom = _bfdot(ex, segv)
        wrow = jnp.broadcast_to(wall[:, u * SEL:(u + 1) * SEL], (ROWS, SEL))
        fac = jnp.where(same_g, wrow / jnp.maximum(denom, 1e-20), 0.0)
        spread = _bfdot(fac, segtv)
        attnw = ex * spread
        colsum = _bfdot(attnw, tcv)
        pool = _bfdot(attnw, st) + _bfdot(colsum, posv)
        z = _bfdot(pool, wvv)
        o1 = jnp.where(keep, z, 0.0).reshape(G, HEADS, HIDDEN).sum(axis=1)
        o1acc[s, u, pl.ds(0, G), :] = o1

    @pl.when(s == NSTEP - 1)
    def _finish():
        o1full = o1acc[...][:, :, 0:G, :].reshape(NQ, HIDDEN)
        htm = _bfdot(o1full, wo_ref[...]) + wob_ref[...]
        out_ref[...] = _bfdot(htm, f2w_ref[...]) + f2b_ref[...]


def _pos_emb():
    freqs = np.arange(0, HIDDEN, 2.0)
    inv = 10000.0 ** (-freqs / HIDDEN)
    seq = np.arange(CHUNK - 1, -1, -1.0)
    si = seq[:, None] * inv[None, :]
    return np.concatenate([np.sin(si), np.cos(si)], axis=-1).astype(np.float32)


@jax.jit
def _run(queries, memories, sq_w, sq_b, sk_w, sk_b, to_q_w, to_kv_w,
         to_out_w, to_out_b, fc2_w, fc2_b):
    q2 = queries[0]
    mem3 = memories[0].reshape(NCHUNK, CHUNK, HIDDEN)
    mem2 = memories[0]
    wv = to_kv_w[:, HIDDEN:]
    pos = jnp.asarray(_pos_emb())

    qk, qkpos, idx8, w8 = pl.pallas_call(
        _score_kernel,
        grid=(PB + 1,),
        in_specs=[
            pl.BlockSpec((NCHUNK, JB, HIDDEN),
                         lambda b: (0, jnp.minimum(b, PB - 1), 0)),
            pl.BlockSpec((NQ, HIDDEN), lambda b: (0, 0)),
            pl.BlockSpec((HIDDEN, HIDDEN), lambda b: (0, 0)),
            pl.BlockSpec((1, HIDDEN), lambda b: (0, 0)),
            pl.BlockSpec((HIDDEN, HIDDEN), lambda b: (0, 0)),
            pl.BlockSpec((1, HIDDEN), lambda b: (0, 0)),
            pl.BlockSpec((HIDDEN, HIDDEN), lambda b: (0, 0)),
            pl.BlockSpec((HIDDEN, HIDDEN), lambda b: (0, 0)),
            pl.BlockSpec((CHUNK, HIDDEN), lambda b: (0, 0)),
        ],
        out_specs=[
            pl.BlockSpec((NQ, HEADS, HIDDEN), lambda b: (0, 0, 0)),
            pl.BlockSpec((NQ, HEADS, CHUNK), lambda b: (0, 0, 0)),
            pl.BlockSpec((NQ, TOPK), lambda b: (0, 0)),
            pl.BlockSpec((NQ, TOPK), lambda b: (0, 0)),
        ],
        scratch_shapes=[pltpu.VMEM((NCHUNK, HIDDEN), jnp.float32)],
        out_shape=[
            jax.ShapeDtypeStruct((NQ, HEADS, HIDDEN), jnp.bfloat16),
            jax.ShapeDtypeStruct((NQ, HEADS, CHUNK), jnp.bfloat16),
            jax.ShapeDtypeStruct((NQ, TOPK), jnp.int32),
            jax.ShapeDtypeStruct((NQ, TOPK), jnp.float32),
        ],
    )(mem3, q2, sq_w, sq_b.reshape(1, HIDDEN), sk_w, sk_b.reshape(1, HIDDEN),
      to_q_w, to_kv_w[:, :HIDDEN], pos)

    qk2 = qk.reshape(NQ * HEADS, HIDDEN)
    qkpos2 = qkpos.reshape(NQ * HEADS, CHUNK)
    idx_flat = idx8.reshape(NQ * TOPK)
    w3 = w8.reshape(NSTEP, 1, U * SEL)

    seg = jnp.asarray(
        (np.arange(TOK)[:, None] // CHUNK == np.arange(SEL)[None, :]
         ).astype(np.float32)).astype(jnp.bfloat16)
    tc = jnp.asarray(
        (np.arange(TOK)[:, None] % CHUNK == np.arange(CHUNK)[None, :]
         ).astype(np.float32)).astype(jnp.bfloat16)
    segt = seg.T
    tct = tc.T
    pos16 = pos.astype(jnp.bfloat16)
    wv16 = wv.astype(jnp.bfloat16)
    wo16 = to_out_w.astype(jnp.bfloat16)
    f2w16 = fc2_w.astype(jnp.bfloat16)

    grid_spec = pltpu.PrefetchScalarGridSpec(
        num_scalar_prefetch=1,
        grid=(NSTEP,),
        in_specs=[
            pl.BlockSpec(memory_space=pl.ANY),
            pl.BlockSpec((U * ROWS, HIDDEN), lambda s, n: (s, 0)),
            pl.BlockSpec((U * ROWS, CHUNK), lambda s, n: (s, 0)),
            pl.BlockSpec((1, 1, U * SEL), lambda s, n: (s, 0, 0)),
            pl.BlockSpec((CHUNK, HIDDEN), lambda s, n: (0, 0)),
            pl.BlockSpec((TOK, SEL), lambda s, n: (0, 0)),
            pl.BlockSpec((SEL, TOK), lambda s, n: (0, 0)),
            pl.BlockSpec((TOK, CHUNK), lambda s, n: (0, 0)),
            pl.BlockSpec((CHUNK, TOK), lambda s, n: (0, 0)),
            pl.BlockSpec((HIDDEN, HIDDEN), lambda s, n: (0, 0)),
            pl.BlockSpec((HIDDEN, HIDDEN), lambda s, n: (0, 0)),
            pl.BlockSpec((1, HIDDEN), lambda s, n: (0, 0)),
            pl.BlockSpec((HIDDEN, 5), lambda s, n: (0, 0)),
            pl.BlockSpec((1, 5), lambda s, n: (0, 0)),
        ],
        out_specs=pl.BlockSpec((NQ, 5), lambda s, n: (0, 0)),
        scratch_shapes=[
            pltpu.VMEM((2, U, TOK, HIDDEN), jnp.float32),
            pltpu.VMEM((NSTEP, U, 8, HIDDEN), jnp.float32),
            pltpu.SemaphoreType.DMA((2, U)),
        ],
    )

    out = pl.pallas_call(
        _attn_kernel,
        grid_spec=grid_spec,
        out_shape=jax.ShapeDtypeStruct((NQ, 5), jnp.float32),
    )(idx_flat, mem2, qk2, qkpos2, w3, pos16, seg, segt, tc, tct, wv16, wo16,
      to_out_b.reshape(1, HIDDEN), f2w16, fc2_b.reshape(1, 5))

    return out.reshape(1, NQ, 5)


def kernel(x, queries, memories, conv1_w, conv1_b, conv2_w, conv2_b,
           fc1_w, fc1_b, sq_w, sq_b, sk_w, sk_b, to_q_w, to_kv_w,
           to_out_w, to_out_b, fc2_w, fc2_b, mask):
    return _run(queries, memories, sq_w, sq_b, sk_w, sk_b, to_q_w, to_kv_w,
                to_out_w, to_out_b, fc2_w, fc2_b)


# speedup vs baseline: 1.1037x; 1.0948x over previous
import numpy as np
import jax
import jax.numpy as jnp
from jax.experimental import pallas as pl
from jax.experimental.pallas import tpu as pltpu

HIDDEN = 512
HEADS = 8
DHEAD = 64
TOPK = 8
CHUNK = 32
NQ = 128
NCHUNK = 625
G = 4
STEPS = NQ // G
ROWS = G * HEADS
SEL = G * TOPK
TOK = SEL * CHUNK
NEG = -3.4028235e38
HIGHEST = jax.lax.Precision.HIGHEST
HIGH = jax.lax.Precision.HIGH


def _bfdot(a, b):
    return jax.lax.dot_general(
        a.astype(jnp.bfloat16), b.astype(jnp.bfloat16),
        (((1,), (0,)), ((), ())), preferred_element_type=jnp.float32)


def _bfdot_t(a, b):
    return jax.lax.dot_general(
        a.astype(jnp.bfloat16), b.astype(jnp.bfloat16),
        (((1,), (1,)), ((), ())), preferred_element_type=jnp.float32)


def _score_kernel(mem3, q2, sqw, sqb, skw, skb, tqw, wkt, pos,
                  qk_out, qkpos_out, idx_out, w_out):
    acc0 = mem3[:, 0, :]
    acc1 = mem3[:, 1, :]
    for j in range(2, CHUNK, 2):
        acc0 = acc0 + mem3[:, j, :]
        acc1 = acc1 + mem3[:, j + 1, :]
    summar = (acc0 + acc1) / (np.float32(CHUNK) + np.float32(1e-5))

    q2v = q2[...]
    sq = _bfdot(q2v, sqw[...]) + sqb[...]
    sk = _bfdot(summar, skw[...]) + skb[...]
    sim = _bfdot_t(sq, sk) * np.float32(HIDDEN ** -0.5)

    iota = jax.lax.broadcasted_iota(jnp.int32, (NQ, NCHUNK), 1)
    s = sim
    logits = []
    idxs = []
    for _ in range(TOPK):
        m = jnp.max(s, axis=1, keepdims=True)
        cand = jnp.where(s >= m, iota, jnp.int32(2 ** 30))
        ix = jnp.min(cand, axis=1, keepdims=True)
        s = jnp.where(iota == ix, jnp.float32(NEG), s)
        logits.append(m)
        idxs.append(ix)
    lg = jnp.concatenate(logits, axis=1)
    e = jnp.exp(lg - lg[:, 0:1])
    w_out[...] = e / jnp.sum(e, axis=1, keepdims=True)
    idx_out[...] = jnp.concatenate(idxs, axis=1)

    qall = _bfdot(q2v, tqw[...]) * np.float32(DHEAD ** -0.5)
    wktv = wkt[...]
    posv = pos[...]
    for h in range(HEADS):
        qk_h = _bfdot(qall[:, h * DHEAD:(h + 1) * DHEAD],
                      wktv[h * DHEAD:(h + 1) * DHEAD, :])
        qk_out[:, h, :] = qk_h
        qkpos_out[:, h, :] = _bfdot_t(qk_h, posv)


def _attn_kernel(idx_ref, memhbm, qk_ref, qkpos_ref, w_ref, pos_ref,
                 seg_ref, segt_ref, tc_ref, tct_ref,
                 wv_ref, wo_ref, wob_ref, f2w_ref, f2b_ref,
                 out_ref, stk, o1acc, gsem):
    s = pl.program_id(0)

    def issue(step, slot):
        for t in range(SEL):
            c = idx_ref[step * SEL + t]
            pltpu.make_async_copy(
                memhbm.at[pl.ds(c * CHUNK, CHUNK), :],
                stk.at[slot, pl.ds(t * CHUNK, CHUNK), :],
                gsem.at[slot]).start()

    @pl.when(s == 0)
    def _first():
        issue(0, 0)

    @pl.when(s < STEPS - 1)
    def _prefetch():
        issue(s + 1, (s + 1) % 2)

    slot = s % 2
    for t in range(SEL):
        pltpu.make_async_copy(
            memhbm.at[pl.ds(0, CHUNK), :],
            stk.at[0, pl.ds(t * CHUNK, CHUNK), :],
            gsem.at[slot]).wait()

    st = stk[slot]
    qk = qk_ref[...]
    segv = seg_ref[...]
    tcv = tc_ref[...]
    qkpos_t = _bfdot(qkpos_ref[...], tct_ref[...])
    scores = _bfdot_t(qk, st) + qkpos_t
    rowmax = jnp.max(scores, axis=1, keepdims=True)
    ex = jnp.exp(scores - rowmax)
    denom = _bfdot(ex, segv)

    ri = jax.lax.broadcasted_iota(jnp.int32, (ROWS, SEL), 0)
    ci = jax.lax.broadcasted_iota(jnp.int32, (ROWS, SEL), 1)
    same_g = (ri // HEADS) == (ci // TOPK)
    wrow = jnp.broadcast_to(w_ref[...].reshape(1, SEL), (ROWS, SEL))
    fac = jnp.where(same_g, wrow / jnp.maximum(denom, 1e-20), 0.0)
    spread = _bfdot(fac, segt_ref[...])
    attnw = ex * spread
    colsum = _bfdot(attnw, tcv)
    pool = (_bfdot(attnw, st)
            + _bfdot(colsum, pos_ref[...]))

    z = _bfdot(pool, wv_ref[...])
    r2 = jax.lax.broadcasted_iota(jnp.int32, (ROWS, HIDDEN), 0)
    c2 = jax.lax.broadcasted_iota(jnp.int32, (ROWS, HIDDEN), 1)
    keep = (r2 % HEADS) == (c2 // DHEAD)
    o1 = jnp.where(keep, z, 0.0).reshape(G, HEADS, HIDDEN).sum(axis=1)
    o1acc[s, pl.ds(0, G), :] = o1

    @pl.when(s == STEPS - 1)
    def _finish():
        o1full = o1acc[...][:, 0:G, :].reshape(NQ, HIDDEN)
        htm = _bfdot(o1full, wo_ref[...]) + wob_ref[...]
        out_ref[...] = _bfdot(htm, f2w_ref[...]) + f2b_ref[...]


def _pos_emb():
    freqs = np.arange(0, HIDDEN, 2.0)
    inv = 10000.0 ** (-freqs / HIDDEN)
    seq = np.arange(CHUNK - 1, -1, -1.0)
    si = seq[:, None] * inv[None, :]
    return np.concatenate([np.sin(si), np.cos(si)], axis=-1).astype(np.float32)


@jax.jit
def _run(queries, memories, sq_w, sq_b, sk_w, sk_b, to_q_w, to_kv_w,
         to_out_w, to_out_b, fc2_w, fc2_b):
    q2 = queries[0]
    mem3 = memories[0].reshape(NCHUNK, CHUNK, HIDDEN)
    mem2 = memories[0]
    wkt = to_kv_w[:, :HIDDEN].T
    wv = to_kv_w[:, HIDDEN:]
    pos = jnp.asarray(_pos_emb())

    qk, qkpos, idx8, w8 = pl.pallas_call(
        _score_kernel,
        out_shape=[
            jax.ShapeDtypeStruct((NQ, HEADS, HIDDEN), jnp.float32),
            jax.ShapeDtypeStruct((NQ, HEADS, CHUNK), jnp.float32),
            jax.ShapeDtypeStruct((NQ, TOPK), jnp.int32),
            jax.ShapeDtypeStruct((NQ, TOPK), jnp.float32),
        ],
    )(mem3, q2, sq_w, sq_b.reshape(1, HIDDEN), sk_w, sk_b.reshape(1, HIDDEN),
      to_q_w, wkt, pos)

    qk2 = qk.reshape(NQ * HEADS, HIDDEN)
    qkpos2 = qkpos.reshape(NQ * HEADS, CHUNK)
    idx_flat = idx8.reshape(NQ * TOPK)
    w3 = w8.reshape(STEPS, 1, SEL)

    seg = jnp.asarray(
        (np.arange(TOK)[:, None] // CHUNK == np.arange(SEL)[None, :]
         ).astype(np.float32))
    tc = jnp.asarray(
        (np.arange(TOK)[:, None] % CHUNK == np.arange(CHUNK)[None, :]
         ).astype(np.float32))
    segt = seg.T
    tct = tc.T

    grid_spec = pltpu.PrefetchScalarGridSpec(
        num_scalar_prefetch=1,
        grid=(STEPS,),
        in_specs=[
            pl.BlockSpec(memory_space=pl.ANY),
            pl.BlockSpec((ROWS, HIDDEN), lambda s, n: (s, 0)),
            pl.BlockSpec((ROWS, CHUNK), lambda s, n: (s, 0)),
            pl.BlockSpec((1, 1, SEL), lambda s, n: (s, 0, 0)),
            pl.BlockSpec((CHUNK, HIDDEN), lambda s, n: (0, 0)),
            pl.BlockSpec((TOK, SEL), lambda s, n: (0, 0)),
            pl.BlockSpec((SEL, TOK), lambda s, n: (0, 0)),
            pl.BlockSpec((TOK, CHUNK), lambda s, n: (0, 0)),
            pl.BlockSpec((CHUNK, TOK), lambda s, n: (0, 0)),
            pl.BlockSpec((HIDDEN, HIDDEN), lambda s, n: (0, 0)),
            pl.BlockSpec((HIDDEN, HIDDEN), lambda s, n: (0, 0)),
            pl.BlockSpec((1, HIDDEN), lambda s, n: (0, 0)),
            pl.BlockSpec((HIDDEN, 5), lambda s, n: (0, 0)),
            pl.BlockSpec((1, 5), lambda s, n: (0, 0)),
        ],
        out_specs=pl.BlockSpec((NQ, 5), lambda s, n: (0, 0)),
        scratch_shapes=[
            pltpu.VMEM((2, TOK, HIDDEN), jnp.float32),
            pltpu.VMEM((STEPS, 8, HIDDEN), jnp.float32),
            pltpu.SemaphoreType.DMA((2,)),
        ],
    )

    out = pl.pallas_call(
        _attn_kernel,
        grid_spec=grid_spec,
        out_shape=jax.ShapeDtypeStruct((NQ, 5), jnp.float32),
    )(idx_flat, mem2, qk2, qkpos2, w3, pos, seg, segt, tc, tct, wv, to_out_w,
      to_out_b.reshape(1, HIDDEN), fc2_w, fc2_b.reshape(1, 5))

    return out.reshape(1, NQ, 5)


def kernel(x, queries, memories, conv1_w, conv1_b, conv2_w, conv2_b,
           fc1_w, fc1_b, sq_w, sq_b, sk_w, sk_b, to_q_w, to_kv_w,
           to_out_w, to_out_b, fc2_w, fc2_b, mask):
    return _run(queries, memories, sq_w, sq_b, sk_w, sk_b, to_q_w, to_kv_w,
                to_out_w, to_out_b, fc2_w, fc2_b)


# speedup vs baseline: 1.1297x; 1.0235x over previous
import numpy as np
import jax
import jax.numpy as jnp
from jax.experimental import pallas as pl
from jax.experimental.pallas import tpu as pltpu

HIDDEN = 512
HEADS = 8
DHEAD = 64
TOPK = 8
CHUNK = 32
NQ = 128
NCHUNK = 625
G = 4
STEPS = NQ // G
ROWS = G * HEADS
SEL = G * TOPK
TOK = SEL * CHUNK
NEG = -3.4028235e38


def _bfdot(a, b):
    return jax.lax.dot_general(
        a.astype(jnp.bfloat16), b.astype(jnp.bfloat16),
        (((1,), (0,)), ((), ())), preferred_element_type=jnp.float32)


def _bfdot_t(a, b):
    return jax.lax.dot_general(
        a.astype(jnp.bfloat16), b.astype(jnp.bfloat16),
        (((1,), (1,)), ((), ())), preferred_element_type=jnp.float32)


CB = 125
PB = NCHUNK // CB


def _score_kernel(mem3, q2, sqw, sqb, skw, skb, tqw, wkt, pos,
                  qk_out, qkpos_out, idx_out, w_out, accr):
    b = pl.program_id(0)

    @pl.when(b < PB)
    def _pool():
        blk = mem3[...]
        acc0 = blk[:, 0, :]
        acc1 = blk[:, 1, :]
        for j in range(2, CHUNK, 2):
            acc0 = acc0 + blk[:, j, :]
            acc1 = acc1 + blk[:, j + 1, :]
        accr[b, pl.ds(0, CB), :] = acc0 + acc1

    @pl.when(b == PB)
    def _rest():
        _score_tail(q2, sqw, sqb, skw, skb, tqw, wkt, pos,
                    qk_out, qkpos_out, idx_out, w_out, accr)


def _score_tail(q2, sqw, sqb, skw, skb, tqw, wkt, pos,
                qk_out, qkpos_out, idx_out, w_out, accr):
    acc = accr[...][:, 0:CB, :].reshape(NCHUNK, HIDDEN)
    summar = acc / (np.float32(CHUNK) + np.float32(1e-5))

    q2v = q2[...]
    sq = _bfdot(q2v, sqw[...]) + sqb[...]
    sk = _bfdot(summar, skw[...]) + skb[...]
    sim = _bfdot_t(sq, sk) * np.float32(HIDDEN ** -0.5)

    iota = jax.lax.broadcasted_iota(jnp.int32, (NQ, NCHUNK), 1)
    s = sim
    logits = []
    idxs = []
    for _ in range(TOPK):
        m = jnp.max(s, axis=1, keepdims=True)
        cand = jnp.where(s >= m, iota, jnp.int32(2 ** 30))
        ix = jnp.min(cand, axis=1, keepdims=True)
        s = jnp.where(iota == ix, jnp.float32(NEG), s)
        logits.append(m)
        idxs.append(ix)
    lg = jnp.concatenate(logits, axis=1)
    e = jnp.exp(lg - lg[:, 0:1])
    w_out[...] = e / jnp.sum(e, axis=1, keepdims=True)
    idx_out[...] = jnp.concatenate(idxs, axis=1)

    qall = _bfdot(q2v, tqw[...]) * np.float32(DHEAD ** -0.5)
    wktv = wkt[...]
    posv = pos[...]
    for h in range(HEADS):
        qk_h = _bfdot(qall[:, h * DHEAD:(h + 1) * DHEAD],
                      wktv[h * DHEAD:(h + 1) * DHEAD, :])
        qk_out[:, h, :] = qk_h
        qkpos_out[:, h, :] = _bfdot_t(qk_h, posv)


def _attn_kernel(idx_ref, memhbm, qk_ref, qkpos_ref, w_ref, pos_ref,
                 seg_ref, segt_ref, tc_ref, tct_ref,
                 wv_ref, wo_ref, wob_ref, f2w_ref, f2b_ref,
                 out_ref, stk, o1acc, gsem):
    s = pl.program_id(0)

    def issue(step, slot):
        for t in range(SEL):
            c = idx_ref[step * SEL + t]
            pltpu.make_async_copy(
                memhbm.at[pl.ds(c * CHUNK, CHUNK), :],
                stk.at[slot, pl.ds(t * CHUNK, CHUNK), :],
                gsem.at[slot]).start()

    @pl.when(s == 0)
    def _first():
        issue(0, 0)

    @pl.when(s < STEPS - 1)
    def _prefetch():
        issue(s + 1, (s + 1) % 2)

    slot = s % 2
    for t in range(SEL):
        pltpu.make_async_copy(
            memhbm.at[pl.ds(0, CHUNK), :],
            stk.at[0, pl.ds(t * CHUNK, CHUNK), :],
            gsem.at[slot]).wait()

    st = stk[slot]
    qk = qk_ref[...]
    segv = seg_ref[...]
    tcv = tc_ref[...]
    qkpos_t = _bfdot(qkpos_ref[...], tct_ref[...])
    scores = _bfdot_t(qk, st) + qkpos_t
    rowmax = jnp.max(scores, axis=1, keepdims=True)
    ex = jnp.exp(scores - rowmax)
    denom = _bfdot(ex, segv)

    ri = jax.lax.broadcasted_iota(jnp.int32, (ROWS, SEL), 0)
    ci = jax.lax.broadcasted_iota(jnp.int32, (ROWS, SEL), 1)
    same_g = (ri // HEADS) == (ci // TOPK)
    wrow = jnp.broadcast_to(w_ref[...].reshape(1, SEL), (ROWS, SEL))
    fac = jnp.where(same_g, wrow / jnp.maximum(denom, 1e-20), 0.0)
    spread = _bfdot(fac, segt_ref[...])
    attnw = ex * spread
    colsum = _bfdot(attnw, tcv)
    pool = (_bfdot(attnw, st)
            + _bfdot(colsum, pos_ref[...]))

    z = _bfdot(pool, wv_ref[...])
    r2 = jax.lax.broadcasted_iota(jnp.int32, (ROWS, HIDDEN), 0)
    c2 = jax.lax.broadcasted_iota(jnp.int32, (ROWS, HIDDEN), 1)
    keep = (r2 % HEADS) == (c2 // DHEAD)
    o1 = jnp.where(keep, z, 0.0).reshape(G, HEADS, HIDDEN).sum(axis=1)
    o1acc[s, pl.ds(0, G), :] = o1

    @pl.when(s == STEPS - 1)
    def _finish():
        o1full = o1acc[...][:, 0:G, :].reshape(NQ, HIDDEN)
        htm = _bfdot(o1full, wo_ref[...]) + wob_ref[...]
        out_ref[...] = _bfdot(htm, f2w_ref[...]) + f2b_ref[...]


def _pos_emb():
    freqs = np.arange(0, HIDDEN, 2.0)
    inv = 10000.0 ** (-freqs / HIDDEN)
    seq = np.arange(CHUNK - 1, -1, -1.0)
    si = seq[:, None] * inv[None, :]
    return np.concatenate([np.sin(si), np.cos(si)], axis=-1).astype(np.float32)


@jax.jit
def _run(queries, memories, sq_w, sq_b, sk_w, sk_b, to_q_w, to_kv_w,
         to_out_w, to_out_b, fc2_w, fc2_b):
    q2 = queries[0]
    mem3 = memories[0].reshape(NCHUNK, CHUNK, HIDDEN)
    mem2 = memories[0]
    wkt = to_kv_w[:, :HIDDEN].T
    wv = to_kv_w[:, HIDDEN:]
    pos = jnp.asarray(_pos_emb())

    qk, qkpos, idx8, w8 = pl.pallas_call(
        _score_kernel,
        grid=(PB + 1,),
        in_specs=[
            pl.BlockSpec((CB, CHUNK, HIDDEN),
                         lambda b: (jnp.minimum(b, PB - 1), 0, 0)),
            pl.BlockSpec((NQ, HIDDEN), lambda b: (0, 0)),
            pl.BlockSpec((HIDDEN, HIDDEN), lambda b: (0, 0)),
            pl.BlockSpec((1, HIDDEN), lambda b: (0, 0)),
            pl.BlockSpec((HIDDEN, HIDDEN), lambda b: (0, 0)),
            pl.BlockSpec((1, HIDDEN), lambda b: (0, 0)),
            pl.BlockSpec((HIDDEN, HIDDEN), lambda b: (0, 0)),
            pl.BlockSpec((HIDDEN, HIDDEN), lambda b: (0, 0)),
            pl.BlockSpec((CHUNK, HIDDEN), lambda b: (0, 0)),
        ],
        out_specs=[
            pl.BlockSpec((NQ, HEADS, HIDDEN), lambda b: (0, 0, 0)),
            pl.BlockSpec((NQ, HEADS, CHUNK), lambda b: (0, 0, 0)),
            pl.BlockSpec((NQ, TOPK), lambda b: (0, 0)),
            pl.BlockSpec((NQ, TOPK), lambda b: (0, 0)),
        ],
        scratch_shapes=[pltpu.VMEM((PB, 128, HIDDEN), jnp.float32)],
        out_shape=[
            jax.ShapeDtypeStruct((NQ, HEADS, HIDDEN), jnp.float32),
            jax.ShapeDtypeStruct((NQ, HEADS, CHUNK), jnp.float32),
            jax.ShapeDtypeStruct((NQ, TOPK), jnp.int32),
            jax.ShapeDtypeStruct((NQ, TOPK), jnp.float32),
        ],
    )(mem3, q2, sq_w, sq_b.reshape(1, HIDDEN), sk_w, sk_b.reshape(1, HIDDEN),
      to_q_w, wkt, pos)

    qk2 = qk.reshape(NQ * HEADS, HIDDEN)
    qkpos2 = qkpos.reshape(NQ * HEADS, CHUNK)
    idx_flat = idx8.reshape(NQ * TOPK)
    w3 = w8.reshape(STEPS, 1, SEL)

    seg = jnp.asarray(
        (np.arange(TOK)[:, None] // CHUNK == np.arange(SEL)[None, :]
         ).astype(np.float32))
    tc = jnp.asarray(
        (np.arange(TOK)[:, None] % CHUNK == np.arange(CHUNK)[None, :]
         ).astype(np.float32))
    segt = seg.T
    tct = tc.T

    grid_spec = pltpu.PrefetchScalarGridSpec(
        num_scalar_prefetch=1,
        grid=(STEPS,),
        in_specs=[
            pl.BlockSpec(memory_space=pl.ANY),
            pl.BlockSpec((ROWS, HIDDEN), lambda s, n: (s, 0)),
            pl.BlockSpec((ROWS, CHUNK), lambda s, n: (s, 0)),
            pl.BlockSpec((1, 1, SEL), lambda s, n: (s, 0, 0)),
            pl.BlockSpec((CHUNK, HIDDEN), lambda s, n: (0, 0)),
            pl.BlockSpec((TOK, SEL), lambda s, n: (0, 0)),
            pl.BlockSpec((SEL, TOK), lambda s, n: (0, 0)),
            pl.BlockSpec((TOK, CHUNK), lambda s, n: (0, 0)),
            pl.BlockSpec((CHUNK, TOK), lambda s, n: (0, 0)),
            pl.BlockSpec((HIDDEN, HIDDEN), lambda s, n: (0, 0)),
            pl.BlockSpec((HIDDEN, HIDDEN), lambda s, n: (0, 0)),
            pl.BlockSpec((1, HIDDEN), lambda s, n: (0, 0)),
            pl.BlockSpec((HIDDEN, 5), lambda s, n: (0, 0)),
            pl.BlockSpec((1, 5), lambda s, n: (0, 0)),
        ],
        out_specs=pl.BlockSpec((NQ, 5), lambda s, n: (0, 0)),
        scratch_shapes=[
            pltpu.VMEM((2, TOK, HIDDEN), jnp.float32),
            pltpu.VMEM((STEPS, 8, HIDDEN), jnp.float32),
            pltpu.SemaphoreType.DMA((2,)),
        ],
    )

    out = pl.pallas_call(
        _attn_kernel,
        grid_spec=grid_spec,
        out_shape=jax.ShapeDtypeStruct((NQ, 5), jnp.float32),
    )(idx_flat, mem2, qk2, qkpos2, w3, pos, seg, segt, tc, tct, wv, to_out_w,
      to_out_b.reshape(1, HIDDEN), fc2_w, fc2_b.reshape(1, 5))

    return out.reshape(1, NQ, 5)


def kernel(x, queries, memories, conv1_w, conv1_b, conv2_w, conv2_b,
           fc1_w, fc1_b, sq_w, sq_b, sk_w, sk_b, to_q_w, to_kv_w,
           to_out_w, to_out_b, fc2_w, fc2_b, mask):
    return _run(queries, memories, sq_w, sq_b, sk_w, sk_b, to_q_w, to_kv_w,
                to_out_w, to_out_b, fc2_w, fc2_b)
